# Initial kernel scaffold; baseline (speedup 1.0000x reference)
import jax
import jax.numpy as jnp
from jax.experimental import pallas as pl


def kernel(dom_x, dom_edge_index, dom_batch, logs_x, logs_batch, W1, b1, W2, b2, W1t, b1t, W2t, b2t, Wlg, blg, Wlt, blt):
    raise NotImplementedError("write your pallas kernel here")


# speedup vs baseline: 15.0614x; 15.0614x over previous
import functools

import jax
import jax.numpy as jnp
from jax import lax
from jax.experimental import pallas as pl
from jax.experimental.pallas import tpu as pltpu
from jax.experimental.pallas import tpu_sc as plsc

N = 50000
E = 800000
B = 64
M = 50000
NBLK = 25
BLK = 2000

NCHUNK = E // 128
DEGT = 3200
DEGP = 16 * DEGT
GROWS = 16 * 3136
QHALF = 1600000
QPAD = QHALF + 128
QT = 100000

_mesh = plsc.VectorSubcoreMesh(core_axis_name="c", subcore_axis_name="s")


def _zero_fill(ref, n):
    def body(i, c):
        ref[pl.ds(i * 16, 16)] = jnp.zeros((16,), jnp.float32)
        return c
    lax.fori_loop(0, n // 16, body, 0)


@functools.partial(
    pl.kernel,
    out_type=[jax.ShapeDtypeStruct((2 * DEGP,), jnp.float32)],
    mesh=_mesh,
    compiler_params=pltpu.CompilerParams(use_tc_tiling_on_sc=False),
    scratch_types=[
        pltpu.VMEM_SHARED((DEGP,), jnp.float32),
        pltpu.VMEM((128,), jnp.int32),
        pltpu.VMEM((128,), jnp.float32),
        pltpu.VMEM((DEGT,), jnp.float32),
    ],
)
def _sc_degree(dst_hbm, deg_hbm, acc, didx, ones, buf):
    cid = lax.axis_index("c")
    sid = lax.axis_index("s")
    wid = cid * 16 + sid
    _zero_fill(buf, DEGT)
    for g in range(8):
        ones[pl.ds(g * 16, 16)] = jnp.full((16,), 1.0, jnp.float32)
    pltpu.sync_copy(buf, acc.at[pl.ds(sid * DEGT, DEGT)])
    plsc.subcore_barrier()
    start = wid * 195 + jnp.minimum(wid, 10)
    nch = 195 + jnp.where(wid < 10, 1, 0)

    def chunk(j, carry):
        off = (start + j) * 128
        pltpu.sync_copy(dst_hbm.at[pl.ds(off, 128)], didx)
        pltpu.sync_copy(ones, acc.at[didx], add=True)
        return carry

    lax.fori_loop(0, nch, chunk, 0)
    plsc.subcore_barrier()
    pltpu.sync_copy(acc.at[pl.ds(sid * DEGT, DEGT)], buf)
    pltpu.sync_copy(buf, deg_hbm.at[pl.ds(cid * DEGP + sid * DEGT, DEGT)])


@functools.partial(
    pl.kernel,
    out_type=[jax.ShapeDtypeStruct((GROWS, 32), jnp.float32),
              jax.ShapeDtypeStruct((GROWS, 32), jnp.float32)],
    mesh=_mesh,
    compiler_params=pltpu.CompilerParams(use_tc_tiling_on_sc=False),
    scratch_types=[
        pltpu.VMEM_SHARED((GROWS, 32), jnp.float32),
        pltpu.VMEM((128,), jnp.int32),
        pltpu.VMEM((128,), jnp.int32),
        pltpu.VMEM((128, 32), jnp.float32),
        pltpu.VMEM((784, 32), jnp.float32),
    ],
)
def _sc_aggregate(src_hbm, dst_hbm, y0_hbm, y1_hbm,
                  g0_hbm, g1_hbm, acc, sidx, didx, rows, buf):
    cid = lax.axis_index("c")
    sid = lax.axis_index("s")

    def zrow(i, c):
        buf[i, pl.ds(0, 16)] = jnp.zeros((16,), jnp.float32)
        buf[i, pl.ds(16, 16)] = jnp.zeros((16,), jnp.float32)
        return c
    lax.fori_loop(0, 784, zrow, 0)

    def zcopy(k, c):
        pltpu.sync_copy(buf, acc.at[pl.ds(sid * 3136 + k * 784, 784)])
        return c
    lax.fori_loop(0, 4, zcopy, 0)
    plsc.subcore_barrier()

    start = sid * 390 + jnp.minimum(sid, 10)
    nch = 390 + jnp.where(sid < 10, 1, 0)

    def chunk(j, carry):
        off = (start + j) * 128
        pltpu.sync_copy(src_hbm.at[pl.ds(off, 128)], sidx)
        pltpu.sync_copy(dst_hbm.at[pl.ds(off, 128)], didx)

        @pl.when(cid == 0)
        def _():
            pltpu.sync_copy(y0_hbm.at[sidx], rows)

        @pl.when(cid == 1)
        def _():
            pltpu.sync_copy(y1_hbm.at[sidx], rows)

        pltpu.sync_copy(rows, acc.at[didx], add=True)
        return carry

    lax.fori_loop(0, nch, chunk, 0)
    plsc.subcore_barrier()

    def drain(k, c):
        r0 = sid * 3136 + k * 784
        pltpu.sync_copy(acc.at[pl.ds(r0, 784)], buf)

        @pl.when(cid == 0)
        def _():
            pltpu.sync_copy(buf, g0_hbm.at[pl.ds(r0, 784)])

        @pl.when(cid == 1)
        def _():
            pltpu.sync_copy(buf, g1_hbm.at[pl.ds(r0, 784)])
        return c
    lax.fori_loop(0, 4, drain, 0)


@functools.partial(
    pl.kernel,
    out_type=[jax.ShapeDtypeStruct((2 * QHALF,), jnp.float32)],
    mesh=_mesh,
    compiler_params=pltpu.CompilerParams(use_tc_tiling_on_sc=False),
    scratch_types=[
        pltpu.VMEM_SHARED((QPAD,), jnp.float32),
        pltpu.VMEM((128,), jnp.int32),
        pltpu.VMEM((128,), jnp.int32),
        pltpu.VMEM((128,), jnp.int32),
        pltpu.VMEM((128,), jnp.int32),
        pltpu.VMEM((128,), jnp.float32),
        pltpu.VMEM((2000,), jnp.float32),
    ],
)
def _sc_pool_matrix(src_hbm, dst_hbm, dinv_hbm, batch_hbm, q_hbm,
                    acc, sidx, didx, bvals, fidx, val, buf):
    cid = lax.axis_index("c")
    sid = lax.axis_index("s")
    wid = cid * 16 + sid
    _zero_fill(buf, 2000)

    def zcopy(k, c):
        pltpu.sync_copy(buf, acc.at[pl.ds(sid * QT + k * 2000, 2000)])
        return c
    lax.fori_loop(0, 50, zcopy, 0)

    @pl.when(wid == 0)
    def _():
        pltpu.sync_copy(buf.at[pl.ds(0, 128)], acc.at[pl.ds(QHALF, 128)])

    plsc.subcore_barrier()

    start = sid * 390 + jnp.minimum(sid, 10)
    nch = 390 + jnp.where(sid < 10, 1, 0)
    base = cid * 25000
    iota16 = lax.iota(jnp.int32, 16)

    def chunk(j, carry):
        off = (start + j) * 128
        pltpu.sync_copy(src_hbm.at[pl.ds(off, 128)], sidx)
        pltpu.sync_copy(dst_hbm.at[pl.ds(off, 128)], didx)
        pltpu.sync_copy(dinv_hbm.at[didx], val)
        pltpu.sync_copy(batch_hbm.at[didx], bvals)
        for g in range(8):
            b = bvals[pl.ds(g * 16, 16)]
            s = sidx[pl.ds(g * 16, 16)]
            sl = s - base
            ok = (sl >= 0) & (sl < 25000)
            flat = jnp.where(ok, sl * 64 + b, QHALF + iota16)
            fidx[pl.ds(g * 16, 16)] = flat
        pltpu.sync_copy(val, acc.at[fidx], add=True)
        return carry

    lax.fori_loop(0, nch, chunk, 0)
    plsc.subcore_barrier()

    def drain(k, c):
        pltpu.sync_copy(acc.at[pl.ds(sid * QT + k * 2000, 2000)], buf)
        pltpu.sync_copy(buf,
                        q_hbm.at[pl.ds(cid * QHALF + sid * QT + k * 2000,
                                       2000)])
        return c
    lax.fori_loop(0, 50, drain, 0)


def _tc_prep_body(dega_ref, degb_ref, x_ref,
                  y0_ref, y1_ref, dinv_ref):
    da = dega_ref[0]
    db = degb_ref[0]
    dv = lax.rsqrt(da + db + 1.0)
    x = x_ref[0]
    y = x * dv
    y0_ref[0] = y[:, :32]
    y1_ref[0] = jnp.concatenate(
        [y[:, 32:], jnp.zeros((BLK, 14), jnp.float32)], axis=1)
    dinv_ref[0] = dv


def _tc_prep(dega, degb, x):
    spec = lambda f: pl.BlockSpec((1, BLK, f), lambda i: (i, 0, 0))
    return pl.pallas_call(
        _tc_prep_body,
        grid=(NBLK,),
        in_specs=[spec(1), spec(1), spec(50)],
        out_specs=[spec(32), spec(32), spec(1)],
        out_shape=[jax.ShapeDtypeStruct((NBLK, BLK, 32), jnp.float32),
                   jax.ShapeDtypeStruct((NBLK, BLK, 32), jnp.float32),
                   jax.ShapeDtypeStruct((NBLK, BLK, 1), jnp.float32)],
    )(dega, degb, x)


def _tc_dense_body(g0_ref, g1_ref, q_ref, x_ref, dinv_ref, batch_ref,
                   lx_ref, lb_ref, w1_ref, b1_ref, w1t_ref, b1t_ref,
                   sg_ref, st_ref, cnt_ref, lsum_ref, cntl_ref):
    i = pl.program_id(0)

    @pl.when(i == 0)
    def _():
        sg_ref[...] = jnp.zeros_like(sg_ref)
        st_ref[...] = jnp.zeros_like(st_ref)
        cnt_ref[...] = jnp.zeros_like(cnt_ref)
        lsum_ref[...] = jnp.zeros_like(lsum_ref)
        cntl_ref[...] = jnp.zeros_like(cntl_ref)

    dv = dinv_ref[0]
    x = x_ref[0]
    xp = jnp.concatenate(
        [x, jnp.zeros((BLK, 14), jnp.float32)], axis=1)
    g = jnp.concatenate([g0_ref[0], g1_ref[0]], axis=1)
    agg = dv * (g + xp * dv)
    hg = jnp.maximum(agg @ w1_ref[...] + b1_ref[...], 0.0)
    ht = jnp.maximum(agg @ w1t_ref[...] + b1t_ref[...], 0.0)
    bcol = batch_ref[0]
    oh = (lax.broadcasted_iota(jnp.int32, (BLK, B), 1) == bcol
          ).astype(jnp.float32)
    pmat = dv * q_ref[0] + oh * (dv * dv)
    dn = (((0,), (0,)), ((), ()))
    f32 = jnp.float32
    sg_ref[...] += lax.dot_general(pmat, hg, dn, preferred_element_type=f32)
    st_ref[...] += lax.dot_general(pmat, ht, dn, preferred_element_type=f32)
    ones1 = jnp.ones((BLK, 1), jnp.float32)
    cnt_ref[...] += lax.dot_general(oh, ones1, dn, preferred_element_type=f32)
    lb = lb_ref[0]
    ohl = (lax.broadcasted_iota(jnp.int32, (BLK, B), 1) == lb
           ).astype(jnp.float32)
    lsum_ref[...] += lax.dot_general(ohl, lx_ref[0], dn,
                                     preferred_element_type=f32)
    cntl_ref[...] += lax.dot_general(ohl, ones1, dn,
                                     preferred_element_type=f32)


def _tc_dense(g0, g1, q, x, dinv, batch, lx, lb, w1p, b1r, w1tp, b1tr):
    spec = lambda f: pl.BlockSpec((1, BLK, f), lambda i: (i, 0, 0))
    wspec = lambda r, c: pl.BlockSpec((r, c), lambda i: (0, 0))
    return pl.pallas_call(
        _tc_dense_body,
        grid=(NBLK,),
        in_specs=[spec(32), spec(32), spec(64), spec(50), spec(1), spec(1),
                  spec(50), spec(1),
                  wspec(64, 64), wspec(1, 64), wspec(64, 64), wspec(1, 64)],
        out_specs=[wspec(B, 64), wspec(B, 64), wspec(B, 1),
                   wspec(B, 50), wspec(B, 1)],
        out_shape=[jax.ShapeDtypeStruct((B, 64), jnp.float32),
                   jax.ShapeDtypeStruct((B, 64), jnp.float32),
                   jax.ShapeDtypeStruct((B, 1), jnp.float32),
                   jax.ShapeDtypeStruct((B, 50), jnp.float32),
                   jax.ShapeDtypeStruct((B, 1), jnp.float32)],
    )(g0, g1, q, x, dinv, batch, lx, lb, w1p, b1r, w1tp, b1tr)


def _tc_final_body(sg_ref, st_ref, cnt_ref, lsum_ref, cntl_ref,
                   w2_ref, b2_ref, w2t_ref, b2t_ref,
                   wlg_ref, blg_ref, wlt_ref, blt_ref, out_ref):
    cnt = cnt_ref[...]
    cm = jnp.maximum(cnt, 1.0)
    pg = (sg_ref[...] @ w2_ref[...] + cnt * b2_ref[...]) / cm
    pt = (st_ref[...] @ w2t_ref[...] + cnt * b2t_ref[...]) / cm
    d1 = pt - pg + 1e-6
    cntl = cntl_ref[...]
    cml = jnp.maximum(cntl, 1.0)
    lsum = lsum_ref[...]
    lg = (lsum @ wlg_ref[...] + cntl * blg_ref[...]) / cml
    lt = (lsum @ wlt_ref[...] + cntl * blt_ref[...]) / cml
    d2 = lt - lg + 1e-6
    out_ref[...] = jnp.reshape(
        (jnp.sum(d1 * d1) + jnp.sum(d2 * d2)) / B, (1, 1))


def _tc_final(sg, st, cnt, lsum, cntl, w2, b2r, w2t, b2tr,
              wlg, blgr, wlt, bltr):
    return pl.pallas_call(
        _tc_final_body,
        out_shape=jax.ShapeDtypeStruct((1, 1), jnp.float32),
    )(sg, st, cnt, lsum, cntl, w2, b2r, w2t, b2tr, wlg, blgr, wlt, bltr)


def kernel(dom_x, dom_edge_index, dom_batch, logs_x, logs_batch,
           W1, b1, W2, b2, W1t, b1t, W2t, b2t, Wlg, blg, Wlt, blt):
    src = dom_edge_index[0].astype(jnp.int32)
    dst = dom_edge_index[1].astype(jnp.int32)
    batch = dom_batch.astype(jnp.int32)
    lb = logs_batch.astype(jnp.int32)

    (degflat,) = _sc_degree(dst)
    dega = degflat[:N].reshape(NBLK, BLK, 1)
    degb = degflat[DEGP:DEGP + N].reshape(NBLK, BLK, 1)

    x_r = dom_x.reshape(NBLK, BLK, 50)
    batch_r = batch.reshape(NBLK, BLK, 1)
    y0_r, y1_r, dinv_r = _tc_prep(dega, degb, x_r)

    g0, g1 = _sc_aggregate(src, dst, y0_r.reshape(N, 32),
                           y1_r.reshape(N, 32))

    (qflat,) = _sc_pool_matrix(src, dst, dinv_r.reshape(N), batch)
    q_r = qflat.reshape(NBLK, BLK, 64)

    w1p = jnp.zeros((64, 64), jnp.float32).at[:50].set(W1)
    w1tp = jnp.zeros((64, 64), jnp.float32).at[:50].set(W1t)
    sg, st, cnt, lsum, cntl = _tc_dense(
        g0[:N].reshape(NBLK, BLK, 32), g1[:N].reshape(NBLK, BLK, 32), q_r,
        x_r, dinv_r, batch_r,
        logs_x.reshape(NBLK, BLK, 50), lb.reshape(NBLK, BLK, 1),
        w1p, b1.reshape(1, 64), w1tp, b1t.reshape(1, 64))

    out = _tc_final(sg, st, cnt, lsum, cntl,
                    W2, b2.reshape(1, 32), W2t, b2t.reshape(1, 32),
                    Wlg, blg.reshape(1, 25), Wlt, blt.reshape(1, 25))
    return out.reshape(())


# speedup vs baseline: 18.4507x; 1.2250x over previous
import functools

import jax
import jax.numpy as jnp
from jax import lax
from jax.experimental import pallas as pl
from jax.experimental.pallas import tpu as pltpu
from jax.experimental.pallas import tpu_sc as plsc

N = 50000
E = 800000
B = 64
M = 50000
NBLK = 25
BLK = 2000

NCHUNK = E // 128
DEGT = 3200
DEGP = 16 * DEGT
GROWS = 16 * 3136
QHALF = 1600000
QPAD = QHALF + 128
QT = 100000

_mesh = plsc.VectorSubcoreMesh(core_axis_name="c", subcore_axis_name="s")


def _zero_fill(ref, n):
    def body(i, c):
        ref[pl.ds(i * 16, 16)] = jnp.zeros((16,), jnp.float32)
        return c
    lax.fori_loop(0, n // 16, body, 0)


@functools.partial(
    pl.kernel,
    out_type=[jax.ShapeDtypeStruct((2 * DEGP,), jnp.float32)],
    mesh=_mesh,
    compiler_params=pltpu.CompilerParams(use_tc_tiling_on_sc=False),
    scratch_types=[
        pltpu.VMEM_SHARED((DEGP,), jnp.float32),
        pltpu.VMEM((128,), jnp.int32),
        pltpu.VMEM((128,), jnp.float32),
        pltpu.VMEM((DEGT,), jnp.float32),
    ],
)
def _sc_degree(dst_hbm, deg_hbm, acc, didx, ones, buf):
    cid = lax.axis_index("c")
    sid = lax.axis_index("s")
    wid = cid * 16 + sid
    _zero_fill(buf, DEGT)
    for g in range(8):
        ones[pl.ds(g * 16, 16)] = jnp.full((16,), 1.0, jnp.float32)
    pltpu.sync_copy(buf, acc.at[pl.ds(sid * DEGT, DEGT)])
    plsc.subcore_barrier()
    start = wid * 195 + jnp.minimum(wid, 10)
    nch = 195 + jnp.where(wid < 10, 1, 0)

    def chunk(j, carry):
        off = (start + j) * 128
        pltpu.sync_copy(dst_hbm.at[pl.ds(off, 128)], didx)
        pltpu.sync_copy(ones, acc.at[didx], add=True)
        return carry

    lax.fori_loop(0, nch, chunk, 0)
    plsc.subcore_barrier()
    pltpu.sync_copy(acc.at[pl.ds(sid * DEGT, DEGT)], buf)
    pltpu.sync_copy(buf, deg_hbm.at[pl.ds(cid * DEGP + sid * DEGT, DEGT)])


@functools.partial(
    pl.kernel,
    out_type=[jax.ShapeDtypeStruct((GROWS, 32), jnp.float32),
              jax.ShapeDtypeStruct((GROWS, 32), jnp.float32)],
    mesh=_mesh,
    compiler_params=pltpu.CompilerParams(use_tc_tiling_on_sc=False),
    scratch_types=[
        pltpu.VMEM_SHARED((GROWS, 32), jnp.float32),
        pltpu.VMEM((128,), jnp.int32),
        pltpu.VMEM((128,), jnp.int32),
        pltpu.VMEM((128, 32), jnp.float32),
        pltpu.VMEM((128,), jnp.int32),
        pltpu.VMEM((128,), jnp.int32),
        pltpu.VMEM((128, 32), jnp.float32),
        pltpu.VMEM((196, 32), jnp.float32),
        pltpu.SemaphoreType.DMA,
        pltpu.SemaphoreType.DMA,
        pltpu.SemaphoreType.DMA,
        pltpu.SemaphoreType.DMA,
        pltpu.SemaphoreType.DMA,
        pltpu.SemaphoreType.DMA,
    ],
)
def _sc_aggregate(src_hbm, dst_hbm, y0_hbm, y1_hbm,
                  g0_hbm, g1_hbm, acc, sidx_a, didx_a, rows_a,
                  sidx_b, didx_b, rows_b, buf,
                  sia, sib, sga, sgb, ssa, ssb):
    cid = lax.axis_index("c")
    sid = lax.axis_index("s")

    def zrow(i, c):
        buf[i, pl.ds(0, 16)] = jnp.zeros((16,), jnp.float32)
        buf[i, pl.ds(16, 16)] = jnp.zeros((16,), jnp.float32)
        return c
    lax.fori_loop(0, 196, zrow, 0)

    def zcopy(k, c):
        pltpu.sync_copy(buf, acc.at[pl.ds(sid * 3136 + k * 196, 196)])
        return c
    lax.fori_loop(0, 16, zcopy, 0)
    plsc.subcore_barrier()

    def gather_start(sidx, rows, sg):
        @pl.when(cid == 0)
        def _():
            pltpu.async_copy(y0_hbm.at[sidx], rows, sg)

        @pl.when(cid == 1)
        def _():
            pltpu.async_copy(y1_hbm.at[sidx], rows, sg)

    start0 = sid * 390

    def chunk2(j, carry):
        @pl.when(j > 0)
        def _():
            pltpu.make_async_copy(rows_a, acc.at[didx_a], ssa).wait()
            pltpu.make_async_copy(rows_b, acc.at[didx_b], ssb).wait()
        offa = (start0 + j * 2) * 128
        offb = offa + 128
        ia1 = pltpu.async_copy(src_hbm.at[pl.ds(offa, 128)], sidx_a, sia)
        ia2 = pltpu.async_copy(dst_hbm.at[pl.ds(offa, 128)], didx_a, sia)
        ib1 = pltpu.async_copy(src_hbm.at[pl.ds(offb, 128)], sidx_b, sib)
        ib2 = pltpu.async_copy(dst_hbm.at[pl.ds(offb, 128)], didx_b, sib)
        ia1.wait()
        ia2.wait()
        gather_start(sidx_a, rows_a, sga)
        ib1.wait()
        ib2.wait()
        gather_start(sidx_b, rows_b, sgb)
        pltpu.make_async_copy(y0_hbm.at[sidx_a], rows_a, sga).wait()
        pltpu.async_copy(rows_a, acc.at[didx_a], ssa, add=True)
        pltpu.make_async_copy(y0_hbm.at[sidx_b], rows_b, sgb).wait()
        pltpu.async_copy(rows_b, acc.at[didx_b], ssb, add=True)
        return carry

    lax.fori_loop(0, 195, chunk2, 0)
    pltpu.make_async_copy(rows_a, acc.at[didx_a], ssa).wait()
    pltpu.make_async_copy(rows_b, acc.at[didx_b], ssb).wait()

    @pl.when(sid < 10)
    def _():
        off = (6240 + sid) * 128
        pltpu.sync_copy(src_hbm.at[pl.ds(off, 128)], sidx_a)
        pltpu.sync_copy(dst_hbm.at[pl.ds(off, 128)], didx_a)

        @pl.when(cid == 0)
        def _():
            pltpu.sync_copy(y0_hbm.at[sidx_a], rows_a)

        @pl.when(cid == 1)
        def _():
            pltpu.sync_copy(y1_hbm.at[sidx_a], rows_a)

        pltpu.sync_copy(rows_a, acc.at[didx_a], add=True)

    plsc.subcore_barrier()

    def drain(k, c):
        r0 = sid * 3136 + k * 196
        pltpu.sync_copy(acc.at[pl.ds(r0, 196)], buf)

        @pl.when(cid == 0)
        def _():
            pltpu.sync_copy(buf, g0_hbm.at[pl.ds(r0, 196)])

        @pl.when(cid == 1)
        def _():
            pltpu.sync_copy(buf, g1_hbm.at[pl.ds(r0, 196)])
        return c
    lax.fori_loop(0, 16, drain, 0)


@functools.partial(
    pl.kernel,
    out_type=[jax.ShapeDtypeStruct((2 * QHALF,), jnp.float32)],
    mesh=_mesh,
    compiler_params=pltpu.CompilerParams(use_tc_tiling_on_sc=False),
    scratch_types=[
        pltpu.VMEM_SHARED((QPAD,), jnp.float32),
        pltpu.VMEM((128,), jnp.int32),
        pltpu.VMEM((128,), jnp.int32),
        pltpu.VMEM((128,), jnp.int32),
        pltpu.VMEM((128,), jnp.int32),
        pltpu.VMEM((128,), jnp.float32),
        pltpu.VMEM((2000,), jnp.float32),
    ],
)
def _sc_pool_matrix(src_hbm, dst_hbm, dinv_hbm, batch_hbm, q_hbm,
                    acc, sidx, didx, bvals, fidx, val, buf):
    cid = lax.axis_index("c")
    sid = lax.axis_index("s")
    wid = cid * 16 + sid
    _zero_fill(buf, 2000)

    def zcopy(k, c):
        pltpu.sync_copy(buf, acc.at[pl.ds(sid * QT + k * 2000, 2000)])
        return c
    lax.fori_loop(0, 50, zcopy, 0)

    @pl.when(wid == 0)
    def _():
        pltpu.sync_copy(buf.at[pl.ds(0, 128)], acc.at[pl.ds(QHALF, 128)])

    plsc.subcore_barrier()

    start = sid * 390 + jnp.minimum(sid, 10)
    nch = 390 + jnp.where(sid < 10, 1, 0)
    base = cid * 25000
    iota16 = lax.iota(jnp.int32, 16)

    def chunk(j, carry):
        off = (start + j) * 128
        pltpu.sync_copy(src_hbm.at[pl.ds(off, 128)], sidx)
        pltpu.sync_copy(dst_hbm.at[pl.ds(off, 128)], didx)
        pltpu.sync_copy(dinv_hbm.at[didx], val)
        pltpu.sync_copy(batch_hbm.at[didx], bvals)
        for g in range(8):
            b = bvals[pl.ds(g * 16, 16)]
            s = sidx[pl.ds(g * 16, 16)]
            sl = s - base
            ok = (sl >= 0) & (sl < 25000)
            flat = jnp.where(ok, sl * 64 + b, QHALF + iota16)
            fidx[pl.ds(g * 16, 16)] = flat
        pltpu.sync_copy(val, acc.at[fidx], add=True)
        return carry

    lax.fori_loop(0, nch, chunk, 0)
    plsc.subcore_barrier()

    def drain(k, c):
        pltpu.sync_copy(acc.at[pl.ds(sid * QT + k * 2000, 2000)], buf)
        pltpu.sync_copy(buf,
                        q_hbm.at[pl.ds(cid * QHALF + sid * QT + k * 2000,
                                       2000)])
        return c
    lax.fori_loop(0, 50, drain, 0)


def _tc_prep_body(dega_ref, degb_ref, x_ref,
                  y0_ref, y1_ref, dinv_ref):
    da = dega_ref[0]
    db = degb_ref[0]
    dv = lax.rsqrt(da + db + 1.0)
    x = x_ref[0]
    y = x * dv
    y0_ref[0] = y[:, :32]
    y1_ref[0] = jnp.concatenate(
        [y[:, 32:], jnp.zeros((BLK, 14), jnp.float32)], axis=1)
    dinv_ref[0] = dv


def _tc_prep(dega, degb, x):
    spec = lambda f: pl.BlockSpec((1, BLK, f), lambda i: (i, 0, 0))
    return pl.pallas_call(
        _tc_prep_body,
        grid=(NBLK,),
        in_specs=[spec(1), spec(1), spec(50)],
        out_specs=[spec(32), spec(32), spec(1)],
        out_shape=[jax.ShapeDtypeStruct((NBLK, BLK, 32), jnp.float32),
                   jax.ShapeDtypeStruct((NBLK, BLK, 32), jnp.float32),
                   jax.ShapeDtypeStruct((NBLK, BLK, 1), jnp.float32)],
    )(dega, degb, x)


def _tc_dense_body(g0_ref, g1_ref, q_ref, x_ref, dinv_ref, batch_ref,
                   lx_ref, lb_ref, w1_ref, b1_ref, w1t_ref, b1t_ref,
                   sg_ref, st_ref, cnt_ref, lsum_ref, cntl_ref):
    i = pl.program_id(0)

    @pl.when(i == 0)
    def _():
        sg_ref[...] = jnp.zeros_like(sg_ref)
        st_ref[...] = jnp.zeros_like(st_ref)
        cnt_ref[...] = jnp.zeros_like(cnt_ref)
        lsum_ref[...] = jnp.zeros_like(lsum_ref)
        cntl_ref[...] = jnp.zeros_like(cntl_ref)

    dv = dinv_ref[0]
    x = x_ref[0]
    xp = jnp.concatenate(
        [x, jnp.zeros((BLK, 14), jnp.float32)], axis=1)
    g = jnp.concatenate([g0_ref[0], g1_ref[0]], axis=1)
    agg = dv * (g + xp * dv)
    hg = jnp.maximum(agg @ w1_ref[...] + b1_ref[...], 0.0)
    ht = jnp.maximum(agg @ w1t_ref[...] + b1t_ref[...], 0.0)
    bcol = batch_ref[0]
    oh = (lax.broadcasted_iota(jnp.int32, (BLK, B), 1) == bcol
          ).astype(jnp.float32)
    pmat = dv * q_ref[0] + oh * (dv * dv)
    dn = (((0,), (0,)), ((), ()))
    f32 = jnp.float32
    sg_ref[...] += lax.dot_general(pmat, hg, dn, preferred_element_type=f32)
    st_ref[...] += lax.dot_general(pmat, ht, dn, preferred_element_type=f32)
    ones1 = jnp.ones((BLK, 1), jnp.float32)
    cnt_ref[...] += lax.dot_general(oh, ones1, dn, preferred_element_type=f32)
    lb = lb_ref[0]
    ohl = (lax.broadcasted_iota(jnp.int32, (BLK, B), 1) == lb
           ).astype(jnp.float32)
    lsum_ref[...] += lax.dot_general(ohl, lx_ref[0], dn,
                                     preferred_element_type=f32)
    cntl_ref[...] += lax.dot_general(ohl, ones1, dn,
                                     preferred_element_type=f32)


def _tc_dense(g0, g1, q, x, dinv, batch, lx, lb, w1p, b1r, w1tp, b1tr):
    spec = lambda f: pl.BlockSpec((1, BLK, f), lambda i: (i, 0, 0))
    wspec = lambda r, c: pl.BlockSpec((r, c), lambda i: (0, 0))
    return pl.pallas_call(
        _tc_dense_body,
        grid=(NBLK,),
        in_specs=[spec(32), spec(32), spec(64), spec(50), spec(1), spec(1),
                  spec(50), spec(1),
                  wspec(64, 64), wspec(1, 64), wspec(64, 64), wspec(1, 64)],
        out_specs=[wspec(B, 64), wspec(B, 64), wspec(B, 1),
                   wspec(B, 50), wspec(B, 1)],
        out_shape=[jax.ShapeDtypeStruct((B, 64), jnp.float32),
                   jax.ShapeDtypeStruct((B, 64), jnp.float32),
                   jax.ShapeDtypeStruct((B, 1), jnp.float32),
                   jax.ShapeDtypeStruct((B, 50), jnp.float32),
                   jax.ShapeDtypeStruct((B, 1), jnp.float32)],
    )(g0, g1, q, x, dinv, batch, lx, lb, w1p, b1r, w1tp, b1tr)


def _tc_final_body(sg_ref, st_ref, cnt_ref, lsum_ref, cntl_ref,
                   w2_ref, b2_ref, w2t_ref, b2t_ref,
                   wlg_ref, blg_ref, wlt_ref, blt_ref, out_ref):
    cnt = cnt_ref[...]
    cm = jnp.maximum(cnt, 1.0)
    pg = (sg_ref[...] @ w2_ref[...] + cnt * b2_ref[...]) / cm
    pt = (st_ref[...] @ w2t_ref[...] + cnt * b2t_ref[...]) / cm
    d1 = pt - pg + 1e-6
    cntl = cntl_ref[...]
    cml = jnp.maximum(cntl, 1.0)
    lsum = lsum_ref[...]
    lg = (lsum @ wlg_ref[...] + cntl * blg_ref[...]) / cml
    lt = (lsum @ wlt_ref[...] + cntl * blt_ref[...]) / cml
    d2 = lt - lg + 1e-6
    out_ref[...] = jnp.reshape(
        (jnp.sum(d1 * d1) + jnp.sum(d2 * d2)) / B, (1, 1))


def _tc_final(sg, st, cnt, lsum, cntl, w2, b2r, w2t, b2tr,
              wlg, blgr, wlt, bltr):
    return pl.pallas_call(
        _tc_final_body,
        out_shape=jax.ShapeDtypeStruct((1, 1), jnp.float32),
    )(sg, st, cnt, lsum, cntl, w2, b2r, w2t, b2tr, wlg, blgr, wlt, bltr)


def kernel(dom_x, dom_edge_index, dom_batch, logs_x, logs_batch,
           W1, b1, W2, b2, W1t, b1t, W2t, b2t, Wlg, blg, Wlt, blt):
    src = dom_edge_index[0].astype(jnp.int32)
    dst = dom_edge_index[1].astype(jnp.int32)
    batch = dom_batch.astype(jnp.int32)
    lb = logs_batch.astype(jnp.int32)

    (degflat,) = _sc_degree(dst)
    dega = degflat[:N].reshape(NBLK, BLK, 1)
    degb = degflat[DEGP:DEGP + N].reshape(NBLK, BLK, 1)

    x_r = dom_x.reshape(NBLK, BLK, 50)
    batch_r = batch.reshape(NBLK, BLK, 1)
    y0_r, y1_r, dinv_r = _tc_prep(dega, degb, x_r)

    g0, g1 = _sc_aggregate(src, dst, y0_r.reshape(N, 32),
                           y1_r.reshape(N, 32))

    (qflat,) = _sc_pool_matrix(src, dst, dinv_r.reshape(N), batch)
    q_r = qflat.reshape(NBLK, BLK, 64)

    w1p = jnp.zeros((64, 64), jnp.float32).at[:50].set(W1)
    w1tp = jnp.zeros((64, 64), jnp.float32).at[:50].set(W1t)
    sg, st, cnt, lsum, cntl = _tc_dense(
        g0[:N].reshape(NBLK, BLK, 32), g1[:N].reshape(NBLK, BLK, 32), q_r,
        x_r, dinv_r, batch_r,
        logs_x.reshape(NBLK, BLK, 50), lb.reshape(NBLK, BLK, 1),
        w1p, b1.reshape(1, 64), w1tp, b1t.reshape(1, 64))

    out = _tc_final(sg, st, cnt, lsum, cntl,
                    W2, b2.reshape(1, 32), W2t, b2t.reshape(1, 32),
                    Wlg, blg.reshape(1, 25), Wlt, blt.reshape(1, 25))
    return out.reshape(())


# speedup vs baseline: 28.7068x; 1.5559x over previous
import functools

import jax
import jax.numpy as jnp
from jax import lax
from jax.experimental import pallas as pl
from jax.experimental.pallas import tpu as pltpu
from jax.experimental.pallas import tpu_sc as plsc

N = 50000
E = 800000
B = 64
M = 50000
NBLK = 25
BLK = 2000

NCHUNK = E // 128
DEGT = 3200
DEGP = 16 * DEGT
GROWS = 16 * 3136
QHALF = 1600000
QPAD = QHALF + 128
QT = 100000

_mesh = plsc.VectorSubcoreMesh(core_axis_name="c", subcore_axis_name="s")


def _zero_fill(ref, n):
    def body(i, c):
        ref[pl.ds(i * 16, 16)] = jnp.zeros((16,), jnp.float32)
        return c
    lax.fori_loop(0, n // 16, body, 0)


@functools.partial(
    pl.kernel,
    out_type=[jax.ShapeDtypeStruct((2 * DEGP,), jnp.float32)],
    mesh=_mesh,
    compiler_params=pltpu.CompilerParams(use_tc_tiling_on_sc=False),
    scratch_types=[
        pltpu.VMEM_SHARED((DEGP,), jnp.float32),
        pltpu.VMEM((128,), jnp.int32),
        pltpu.VMEM((128,), jnp.float32),
        pltpu.VMEM((DEGT,), jnp.float32),
    ],
)
def _sc_degree(dst_hbm, deg_hbm, acc, didx, ones, buf):
    cid = lax.axis_index("c")
    sid = lax.axis_index("s")
    wid = cid * 16 + sid
    _zero_fill(buf, DEGT)
    for g in range(8):
        ones[pl.ds(g * 16, 16)] = jnp.full((16,), 1.0, jnp.float32)
    pltpu.sync_copy(buf, acc.at[pl.ds(sid * DEGT, DEGT)])
    plsc.subcore_barrier()
    start = wid * 195 + jnp.minimum(wid, 10)
    nch = 195 + jnp.where(wid < 10, 1, 0)

    def chunk(j, carry):
        off = (start + j) * 128
        pltpu.sync_copy(dst_hbm.at[pl.ds(off, 128)], didx)
        pltpu.sync_copy(ones, acc.at[didx], add=True)
        return carry

    lax.fori_loop(0, nch, chunk, 0)
    plsc.subcore_barrier()
    pltpu.sync_copy(acc.at[pl.ds(sid * DEGT, DEGT)], buf)
    pltpu.sync_copy(buf, deg_hbm.at[pl.ds(cid * DEGP + sid * DEGT, DEGT)])


@functools.partial(
    pl.kernel,
    out_type=[jax.ShapeDtypeStruct((GROWS, 32), jnp.float32),
              jax.ShapeDtypeStruct((GROWS, 32), jnp.float32)],
    mesh=_mesh,
    compiler_params=pltpu.CompilerParams(use_tc_tiling_on_sc=False),
    scratch_types=[
        pltpu.VMEM_SHARED((GROWS, 32), jnp.float32),
        pltpu.VMEM((128,), jnp.int32),
        pltpu.VMEM((128,), jnp.int32),
        pltpu.VMEM((128, 32), jnp.float32),
        pltpu.VMEM((128,), jnp.int32),
        pltpu.VMEM((128,), jnp.int32),
        pltpu.VMEM((128, 32), jnp.float32),
        pltpu.VMEM((196, 32), jnp.float32),
        pltpu.SemaphoreType.DMA,
        pltpu.SemaphoreType.DMA,
        pltpu.SemaphoreType.DMA,
        pltpu.SemaphoreType.DMA,
        pltpu.SemaphoreType.DMA,
        pltpu.SemaphoreType.DMA,
    ],
)
def _sc_aggregate(src_hbm, dst_hbm, y0_hbm, y1_hbm,
                  g0_hbm, g1_hbm, acc, sidx_a, didx_a, rows_a,
                  sidx_b, didx_b, rows_b, buf,
                  sia, sib, sga, sgb, ssa, ssb):
    cid = lax.axis_index("c")
    sid = lax.axis_index("s")

    def zrow(i, c):
        buf[i, pl.ds(0, 16)] = jnp.zeros((16,), jnp.float32)
        buf[i, pl.ds(16, 16)] = jnp.zeros((16,), jnp.float32)
        return c
    lax.fori_loop(0, 196, zrow, 0)

    def zcopy(k, c):
        pltpu.sync_copy(buf, acc.at[pl.ds(sid * 3136 + k * 196, 196)])
        return c
    lax.fori_loop(0, 16, zcopy, 0)
    plsc.subcore_barrier()

    def gather_start(sidx, rows, sg):
        @pl.when(cid == 0)
        def _():
            pltpu.async_copy(y0_hbm.at[sidx], rows, sg)

        @pl.when(cid == 1)
        def _():
            pltpu.async_copy(y1_hbm.at[sidx], rows, sg)

    start0 = sid * 390

    def chunk2(j, carry):
        @pl.when(j > 0)
        def _():
            pltpu.make_async_copy(rows_a, acc.at[didx_a], ssa).wait()
            pltpu.make_async_copy(rows_b, acc.at[didx_b], ssb).wait()
        offa = (start0 + j * 2) * 128
        offb = offa + 128
        ia1 = pltpu.async_copy(src_hbm.at[pl.ds(offa, 128)], sidx_a, sia)
        ia2 = pltpu.async_copy(dst_hbm.at[pl.ds(offa, 128)], didx_a, sia)
        ib1 = pltpu.async_copy(src_hbm.at[pl.ds(offb, 128)], sidx_b, sib)
        ib2 = pltpu.async_copy(dst_hbm.at[pl.ds(offb, 128)], didx_b, sib)
        ia1.wait()
        ia2.wait()
        gather_start(sidx_a, rows_a, sga)
        ib1.wait()
        ib2.wait()
        gather_start(sidx_b, rows_b, sgb)
        pltpu.make_async_copy(y0_hbm.at[sidx_a], rows_a, sga).wait()
        pltpu.async_copy(rows_a, acc.at[didx_a], ssa, add=True)
        pltpu.make_async_copy(y0_hbm.at[sidx_b], rows_b, sgb).wait()
        pltpu.async_copy(rows_b, acc.at[didx_b], ssb, add=True)
        return carry

    lax.fori_loop(0, 195, chunk2, 0)
    pltpu.make_async_copy(rows_a, acc.at[didx_a], ssa).wait()
    pltpu.make_async_copy(rows_b, acc.at[didx_b], ssb).wait()

    @pl.when(sid < 10)
    def _():
        off = (6240 + sid) * 128
        pltpu.sync_copy(src_hbm.at[pl.ds(off, 128)], sidx_a)
        pltpu.sync_copy(dst_hbm.at[pl.ds(off, 128)], didx_a)

        @pl.when(cid == 0)
        def _():
            pltpu.sync_copy(y0_hbm.at[sidx_a], rows_a)

        @pl.when(cid == 1)
        def _():
            pltpu.sync_copy(y1_hbm.at[sidx_a], rows_a)

        pltpu.sync_copy(rows_a, acc.at[didx_a], add=True)

    plsc.subcore_barrier()

    def drain(k, c):
        r0 = sid * 3136 + k * 196
        pltpu.sync_copy(acc.at[pl.ds(r0, 196)], buf)

        @pl.when(cid == 0)
        def _():
            pltpu.sync_copy(buf, g0_hbm.at[pl.ds(r0, 196)])

        @pl.when(cid == 1)
        def _():
            pltpu.sync_copy(buf, g1_hbm.at[pl.ds(r0, 196)])
        return c
    lax.fori_loop(0, 16, drain, 0)


@functools.partial(
    pl.kernel,
    out_type=[jax.ShapeDtypeStruct((2 * QHALF,), jnp.float32)],
    mesh=_mesh,
    compiler_params=pltpu.CompilerParams(use_tc_tiling_on_sc=False),
    scratch_types=[
        pltpu.VMEM_SHARED((QPAD,), jnp.float32),
        pltpu.VMEM((128,), jnp.int32),
        pltpu.VMEM((128,), jnp.int32),
        pltpu.VMEM((128,), jnp.int32),
        pltpu.VMEM((128,), jnp.int32),
        pltpu.VMEM((128,), jnp.float32),
        pltpu.VMEM((128,), jnp.int32),
        pltpu.VMEM((128,), jnp.int32),
        pltpu.VMEM((128,), jnp.int32),
        pltpu.VMEM((128,), jnp.int32),
        pltpu.VMEM((128,), jnp.float32),
        pltpu.VMEM((2000,), jnp.float32),
        pltpu.SemaphoreType.DMA,
        pltpu.SemaphoreType.DMA,
        pltpu.SemaphoreType.DMA,
        pltpu.SemaphoreType.DMA,
        pltpu.SemaphoreType.DMA,
        pltpu.SemaphoreType.DMA,
    ],
)
def _sc_pool_matrix(src_hbm, dst_hbm, dinv_hbm, batch_hbm, q_hbm,
                    acc, sidx_a, didx_a, bvals_a, fidx_a, val_a,
                    sidx_b, didx_b, bvals_b, fidx_b, val_b, buf,
                    sia, sib, sga, sgb, ssa, ssb):
    cid = lax.axis_index("c")
    sid = lax.axis_index("s")
    wid = cid * 16 + sid
    _zero_fill(buf, 2000)

    def zcopy(k, c):
        pltpu.sync_copy(buf, acc.at[pl.ds(sid * QT + k * 2000, 2000)])
        return c
    lax.fori_loop(0, 50, zcopy, 0)

    @pl.when(wid == 0)
    def _():
        pltpu.sync_copy(buf.at[pl.ds(0, 128)], acc.at[pl.ds(QHALF, 128)])

    plsc.subcore_barrier()

    base = cid * 25000
    iota16 = lax.iota(jnp.int32, 16)
    start0 = sid * 390

    def make_fidx(sidx, bvals, fidx):
        for g in range(8):
            b = bvals[pl.ds(g * 16, 16)]
            s = sidx[pl.ds(g * 16, 16)]
            sl = s - base
            ok = (sl >= 0) & (sl < 25000)
            flat = jnp.where(ok, sl * 64 + b, QHALF + iota16)
            fidx[pl.ds(g * 16, 16)] = flat

    def chunk2(j, carry):
        @pl.when(j > 0)
        def _():
            pltpu.make_async_copy(val_a, acc.at[fidx_a], ssa).wait()
            pltpu.make_async_copy(val_b, acc.at[fidx_b], ssb).wait()
        offa = (start0 + j * 2) * 128
        offb = offa + 128
        ia1 = pltpu.async_copy(src_hbm.at[pl.ds(offa, 128)], sidx_a, sia)
        ia2 = pltpu.async_copy(dst_hbm.at[pl.ds(offa, 128)], didx_a, sia)
        ib1 = pltpu.async_copy(src_hbm.at[pl.ds(offb, 128)], sidx_b, sib)
        ib2 = pltpu.async_copy(dst_hbm.at[pl.ds(offb, 128)], didx_b, sib)
        ia1.wait()
        ia2.wait()
        pltpu.async_copy(dinv_hbm.at[didx_a], val_a, sga)
        pltpu.async_copy(batch_hbm.at[didx_a], bvals_a, sga)
        ib1.wait()
        ib2.wait()
        pltpu.async_copy(dinv_hbm.at[didx_b], val_b, sgb)
        pltpu.async_copy(batch_hbm.at[didx_b], bvals_b, sgb)
        pltpu.make_async_copy(dinv_hbm.at[didx_a], val_a, sga).wait()
        pltpu.make_async_copy(batch_hbm.at[didx_a], bvals_a, sga).wait()
        make_fidx(sidx_a, bvals_a, fidx_a)
        pltpu.async_copy(val_a, acc.at[fidx_a], ssa, add=True)
        pltpu.make_async_copy(dinv_hbm.at[didx_b], val_b, sgb).wait()
        pltpu.make_async_copy(batch_hbm.at[didx_b], bvals_b, sgb).wait()
        make_fidx(sidx_b, bvals_b, fidx_b)
        pltpu.async_copy(val_b, acc.at[fidx_b], ssb, add=True)
        return carry

    lax.fori_loop(0, 195, chunk2, 0)
    pltpu.make_async_copy(val_a, acc.at[fidx_a], ssa).wait()
    pltpu.make_async_copy(val_b, acc.at[fidx_b], ssb).wait()

    @pl.when(sid < 10)
    def _():
        off = (6240 + sid) * 128
        pltpu.sync_copy(src_hbm.at[pl.ds(off, 128)], sidx_a)
        pltpu.sync_copy(dst_hbm.at[pl.ds(off, 128)], didx_a)
        pltpu.sync_copy(dinv_hbm.at[didx_a], val_a)
        pltpu.sync_copy(batch_hbm.at[didx_a], bvals_a)
        make_fidx(sidx_a, bvals_a, fidx_a)
        pltpu.sync_copy(val_a, acc.at[fidx_a], add=True)

    plsc.subcore_barrier()

    def drain(k, c):
        pltpu.sync_copy(acc.at[pl.ds(sid * QT + k * 2000, 2000)], buf)
        pltpu.sync_copy(buf,
                        q_hbm.at[pl.ds(cid * QHALF + sid * QT + k * 2000,
                                       2000)])
        return c
    lax.fori_loop(0, 50, drain, 0)


def _tc_prep_body(dega_ref, degb_ref, x_ref,
                  y0_ref, y1_ref, dinv_ref):
    da = dega_ref[0]
    db = degb_ref[0]
    dv = lax.rsqrt(da + db + 1.0)
    x = x_ref[0]
    y = x * dv
    y0_ref[0] = y[:, :32]
    y1_ref[0] = jnp.concatenate(
        [y[:, 32:], jnp.zeros((BLK, 14), jnp.float32)], axis=1)
    dinv_ref[0] = dv


def _tc_prep(dega, degb, x):
    spec = lambda f: pl.BlockSpec((1, BLK, f), lambda i: (i, 0, 0))
    return pl.pallas_call(
        _tc_prep_body,
        grid=(NBLK,),
        in_specs=[spec(1), spec(1), spec(50)],
        out_specs=[spec(32), spec(32), spec(1)],
        out_shape=[jax.ShapeDtypeStruct((NBLK, BLK, 32), jnp.float32),
                   jax.ShapeDtypeStruct((NBLK, BLK, 32), jnp.float32),
                   jax.ShapeDtypeStruct((NBLK, BLK, 1), jnp.float32)],
    )(dega, degb, x)


def _tc_dense_body(g0_ref, g1_ref, q_ref, x_ref, dinv_ref, batch_ref,
                   lx_ref, lb_ref, w1_ref, b1_ref, w1t_ref, b1t_ref,
                   sg_ref, st_ref, cnt_ref, lsum_ref, cntl_ref):
    i = pl.program_id(0)

    @pl.when(i == 0)
    def _():
        sg_ref[...] = jnp.zeros_like(sg_ref)
        st_ref[...] = jnp.zeros_like(st_ref)
        cnt_ref[...] = jnp.zeros_like(cnt_ref)
        lsum_ref[...] = jnp.zeros_like(lsum_ref)
        cntl_ref[...] = jnp.zeros_like(cntl_ref)

    dv = dinv_ref[0]
    x = x_ref[0]
    xp = jnp.concatenate(
        [x, jnp.zeros((BLK, 14), jnp.float32)], axis=1)
    g = jnp.concatenate([g0_ref[0], g1_ref[0]], axis=1)
    agg = dv * (g + xp * dv)
    hg = jnp.maximum(agg @ w1_ref[...] + b1_ref[...], 0.0)
    ht = jnp.maximum(agg @ w1t_ref[...] + b1t_ref[...], 0.0)
    bcol = batch_ref[0]
    oh = (lax.broadcasted_iota(jnp.int32, (BLK, B), 1) == bcol
          ).astype(jnp.float32)
    pmat = dv * q_ref[0] + oh * (dv * dv)
    dn = (((0,), (0,)), ((), ()))
    f32 = jnp.float32
    sg_ref[...] += lax.dot_general(pmat, hg, dn, preferred_element_type=f32)
    st_ref[...] += lax.dot_general(pmat, ht, dn, preferred_element_type=f32)
    ones1 = jnp.ones((BLK, 1), jnp.float32)
    cnt_ref[...] += lax.dot_general(oh, ones1, dn, preferred_element_type=f32)
    lb = lb_ref[0]
    ohl = (lax.broadcasted_iota(jnp.int32, (BLK, B), 1) == lb
           ).astype(jnp.float32)
    lsum_ref[...] += lax.dot_general(ohl, lx_ref[0], dn,
                                     preferred_element_type=f32)
    cntl_ref[...] += lax.dot_general(ohl, ones1, dn,
                                     preferred_element_type=f32)


def _tc_dense(g0, g1, q, x, dinv, batch, lx, lb, w1p, b1r, w1tp, b1tr):
    spec = lambda f: pl.BlockSpec((1, BLK, f), lambda i: (i, 0, 0))
    wspec = lambda r, c: pl.BlockSpec((r, c), lambda i: (0, 0))
    return pl.pallas_call(
        _tc_dense_body,
        grid=(NBLK,),
        in_specs=[spec(32), spec(32), spec(64), spec(50), spec(1), spec(1),
                  spec(50), spec(1),
                  wspec(64, 64), wspec(1, 64), wspec(64, 64), wspec(1, 64)],
        out_specs=[wspec(B, 64), wspec(B, 64), wspec(B, 1),
                   wspec(B, 50), wspec(B, 1)],
        out_shape=[jax.ShapeDtypeStruct((B, 64), jnp.float32),
                   jax.ShapeDtypeStruct((B, 64), jnp.float32),
                   jax.ShapeDtypeStruct((B, 1), jnp.float32),
                   jax.ShapeDtypeStruct((B, 50), jnp.float32),
                   jax.ShapeDtypeStruct((B, 1), jnp.float32)],
    )(g0, g1, q, x, dinv, batch, lx, lb, w1p, b1r, w1tp, b1tr)


def _tc_final_body(sg_ref, st_ref, cnt_ref, lsum_ref, cntl_ref,
                   w2_ref, b2_ref, w2t_ref, b2t_ref,
                   wlg_ref, blg_ref, wlt_ref, blt_ref, out_ref):
    cnt = cnt_ref[...]
    cm = jnp.maximum(cnt, 1.0)
    pg = (sg_ref[...] @ w2_ref[...] + cnt * b2_ref[...]) / cm
    pt = (st_ref[...] @ w2t_ref[...] + cnt * b2t_ref[...]) / cm
    d1 = pt - pg + 1e-6
    cntl = cntl_ref[...]
    cml = jnp.maximum(cntl, 1.0)
    lsum = lsum_ref[...]
    lg = (lsum @ wlg_ref[...] + cntl * blg_ref[...]) / cml
    lt = (lsum @ wlt_ref[...] + cntl * blt_ref[...]) / cml
    d2 = lt - lg + 1e-6
    out_ref[...] = jnp.reshape(
        (jnp.sum(d1 * d1) + jnp.sum(d2 * d2)) / B, (1, 1))


def _tc_final(sg, st, cnt, lsum, cntl, w2, b2r, w2t, b2tr,
              wlg, blgr, wlt, bltr):
    return pl.pallas_call(
        _tc_final_body,
        out_shape=jax.ShapeDtypeStruct((1, 1), jnp.float32),
    )(sg, st, cnt, lsum, cntl, w2, b2r, w2t, b2tr, wlg, blgr, wlt, bltr)


def kernel(dom_x, dom_edge_index, dom_batch, logs_x, logs_batch,
           W1, b1, W2, b2, W1t, b1t, W2t, b2t, Wlg, blg, Wlt, blt):
    src = dom_edge_index[0].astype(jnp.int32)
    dst = dom_edge_index[1].astype(jnp.int32)
    batch = dom_batch.astype(jnp.int32)
    lb = logs_batch.astype(jnp.int32)

    (degflat,) = _sc_degree(dst)
    dega = degflat[:N].reshape(NBLK, BLK, 1)
    degb = degflat[DEGP:DEGP + N].reshape(NBLK, BLK, 1)

    x_r = dom_x.reshape(NBLK, BLK, 50)
    batch_r = batch.reshape(NBLK, BLK, 1)
    y0_r, y1_r, dinv_r = _tc_prep(dega, degb, x_r)

    g0, g1 = _sc_aggregate(src, dst, y0_r.reshape(N, 32),
                           y1_r.reshape(N, 32))

    (qflat,) = _sc_pool_matrix(src, dst, dinv_r.reshape(N), batch)
    q_r = qflat.reshape(NBLK, BLK, 64)

    w1p = jnp.zeros((64, 64), jnp.float32).at[:50].set(W1)
    w1tp = jnp.zeros((64, 64), jnp.float32).at[:50].set(W1t)
    sg, st, cnt, lsum, cntl = _tc_dense(
        g0[:N].reshape(NBLK, BLK, 32), g1[:N].reshape(NBLK, BLK, 32), q_r,
        x_r, dinv_r, batch_r,
        logs_x.reshape(NBLK, BLK, 50), lb.reshape(NBLK, BLK, 1),
        w1p, b1.reshape(1, 64), w1tp, b1t.reshape(1, 64))

    out = _tc_final(sg, st, cnt, lsum, cntl,
                    W2, b2.reshape(1, 32), W2t, b2t.reshape(1, 32),
                    Wlg, blg.reshape(1, 25), Wlt, blt.reshape(1, 25))
    return out.reshape(())
